# baseline (speedup 1.0000x reference)
import functools

import jax
import jax.numpy as jnp
from jax import lax
from jax.experimental import pallas as pl
from jax.experimental.pallas import tpu as pltpu
from jax.experimental.pallas import tpu_sc as plsc


def kernel(x, table):
    B, H = x.shape
    V, D = table.shape
    total = B * H

    info = plsc.get_sparse_core_info()
    NC = info.num_cores
    NW = NC * info.num_subcores
    b_per_w = total // NW
    CHUNK = 128
    n_chunks = b_per_w // CHUNK
    K = 4
    n_outer = n_chunks // K
    BLK = K * CHUNK

    idx = x.reshape(NW, n_chunks, CHUNK)

    mesh = plsc.VectorSubcoreMesh(core_axis_name="c", subcore_axis_name="s")

    @functools.partial(
        pl.kernel,
        mesh=mesh,
        out_type=jax.ShapeDtypeStruct((total, D), table.dtype),
        compiler_params=pltpu.CompilerParams(use_tc_tiling_on_sc=False),
        scratch_types=[
            pltpu.VMEM((n_chunks, CHUNK), jnp.int32),
            pltpu.VMEM((BLK, D), jnp.float32),
            pltpu.SemaphoreType.DMA,
        ],
    )
    def emb_kernel(idx_hbm, table_hbm, out_hbm, idx_v, rows_v, sem):
        wid = lax.axis_index("s") * NC + lax.axis_index("c")
        pltpu.sync_copy(idx_hbm.at[wid], idx_v)
        base = wid * b_per_w

        def body(it, carry):
            copies = []
            for j in range(K):
                c = it * K + j
                copies.append(pltpu.async_copy(
                    table_hbm.at[idx_v.at[c]],
                    rows_v.at[pl.ds(j * CHUNK, CHUNK)],
                    sem,
                ))
            for cp in copies:
                cp.wait()
            pltpu.sync_copy(rows_v, out_hbm.at[pl.ds(base + it * BLK, BLK)])
            return carry

        lax.fori_loop(0, n_outer, body, None)

    out = emb_kernel(idx, table)
    return out.reshape(B, H, D)


# speedup vs baseline: 1.0258x; 1.0258x over previous
import functools

import jax
import jax.numpy as jnp
from jax import lax
from jax.experimental import pallas as pl
from jax.experimental.pallas import tpu as pltpu
from jax.experimental.pallas import tpu_sc as plsc


def kernel(x, table):
    B, H = x.shape
    V, D = table.shape
    total = B * H

    info = plsc.get_sparse_core_info()
    NC = info.num_cores
    NW = NC * info.num_subcores
    b_per_w = total // NW
    CHUNK = 128
    n_chunks = b_per_w // CHUNK
    K = 2
    BLK = K * CHUNK
    NBUF = 4
    n_outer = n_chunks // K
    n_groups = n_outer // NBUF

    idx = x.reshape(NW, n_chunks, CHUNK)

    mesh = plsc.VectorSubcoreMesh(core_axis_name="c", subcore_axis_name="s")

    @functools.partial(
        pl.kernel,
        mesh=mesh,
        out_type=jax.ShapeDtypeStruct((total, D), table.dtype),
        compiler_params=pltpu.CompilerParams(use_tc_tiling_on_sc=False),
        scratch_types=[
            pltpu.VMEM((n_chunks, CHUNK), jnp.int32),
            pltpu.VMEM((NBUF * BLK, D), jnp.float32),
        ] + [pltpu.SemaphoreType.DMA] * (2 * NBUF),
    )
    def emb_kernel(idx_hbm, table_hbm, out_hbm, idx_v, rows_v, *sems):
        sem_g = sems[:NBUF]
        sem_w = sems[NBUF:]
        wid = lax.axis_index("s") * NC + lax.axis_index("c")
        pltpu.sync_copy(idx_hbm.at[wid], idx_v)
        base = wid * b_per_w

        def fire_gather(it, b):
            for j in range(K):
                pltpu.async_copy(
                    table_hbm.at[idx_v.at[it * K + j]],
                    rows_v.at[pl.ds(b * BLK + j * CHUNK, CHUNK)],
                    sem_g[b],
                )

        def wait_gather(b):
            for j in range(K):
                pltpu.make_async_copy(
                    table_hbm.at[idx_v.at[0]],
                    rows_v.at[pl.ds(b * BLK + j * CHUNK, CHUNK)],
                    sem_g[b],
                ).wait()

        def start_write(b, it):
            pltpu.async_copy(
                rows_v.at[pl.ds(b * BLK, BLK)],
                out_hbm.at[pl.ds(base + it * BLK, BLK)],
                sem_w[b],
            )

        def wait_write(b):
            pltpu.make_async_copy(
                rows_v.at[pl.ds(b * BLK, BLK)],
                out_hbm.at[pl.ds(0, BLK)],
                sem_w[b],
            ).wait()

        for b in range(NBUF):
            fire_gather(b, b)

        def body(q, carry):
            for b in range(NBUF):
                it = q * NBUF + b
                wait_gather(b)
                start_write(b, it)

                @pl.when(q < n_groups - 1)
                def _():
                    wait_write(b)
                    fire_gather(it + NBUF, b)

            return carry

        lax.fori_loop(0, n_groups, body, None)
        for b in range(NBUF):
            wait_write(b)

    out = emb_kernel(idx, table)
    return out.reshape(B, H, D)
